# baseline (speedup 1.0000x reference)
import functools

import jax
import jax.numpy as jnp
from jax import lax
from jax.experimental import pallas as pl
from jax.experimental.pallas import tpu as pltpu
from jax.experimental.pallas import tpu_sc as plsc

NUM_HIDDENS = 768
LANES = 16
SEG_SLICES = NUM_HIDDENS // LANES
NC, NS = 2, 16
NW = NC * NS
ROWS = 4 * 8192

SC_ROWS = 8192
SC_ROWS_PER_W = SC_ROWS // NW
CHUNK = 32
NBUF = 4
NCHUNKS = SC_ROWS_PER_W // CHUNK

TC_ROWS = ROWS - SC_ROWS
TC_BLK = 1024
TC_OFF_BLKS = SC_ROWS // TC_BLK


def _sc_add(xf, seg2, flag):
    mesh = plsc.VectorSubcoreMesh(core_axis_name="c", subcore_axis_name="s")

    @functools.partial(
        pl.kernel,
        mesh=mesh,
        out_type=jax.ShapeDtypeStruct((SC_ROWS, NUM_HIDDENS), jnp.float32),
        scratch_types=[
            pltpu.VMEM((2, NUM_HIDDENS), jnp.float32),
            pltpu.VMEM((LANES,), jnp.int32),
        ] + [pltpu.VMEM((CHUNK, NUM_HIDDENS), jnp.float32)] * NBUF
          + [pltpu.SemaphoreType.DMA] * (2 * NBUF),
    )
    def k(x_hbm, seg_hbm, flag_hbm, out_hbm, seg_v, flag_v, *ring):
        bufs = ring[:NBUF]
        in_sems = ring[NBUF:2 * NBUF]
        out_sems = ring[2 * NBUF:]
        wid = lax.axis_index("s") * NC + lax.axis_index("c")
        pltpu.sync_copy(seg_hbm, seg_v)
        pltpu.sync_copy(flag_hbm, flag_v)
        f = flag_v[...] != 0
        segs = [
            jnp.where(f, seg_v[0, pl.ds(j * LANES, LANES)],
                      seg_v[1, pl.ds(j * LANES, LANES)])
            for j in range(SEG_SLICES)
        ]
        row0 = wid * SC_ROWS_PER_W

        def in_copy(g):
            b = g % NBUF
            return pltpu.make_async_copy(
                x_hbm.at[pl.ds(row0 + g * CHUNK, CHUNK)], bufs[b], in_sems[b])

        def out_copy(g):
            b = g % NBUF
            return pltpu.make_async_copy(
                bufs[b], out_hbm.at[pl.ds(row0 + g * CHUNK, CHUNK)], out_sems[b])

        def compute(g):
            buf = bufs[g % NBUF]

            def row_body(r, c):
                for j in range(SEG_SLICES):
                    sl = pl.ds(j * LANES, LANES)
                    plsc.addupdate(buf.at[r, sl], segs[j])
                return c

            lax.fori_loop(0, CHUNK, row_body, 0)

        for g in range(NBUF - 1):
            in_copy(g).start()
        for g in range(NCHUNKS):
            in_copy(g).wait()
            compute(g)
            out_copy(g).start()
            nxt = g + NBUF - 1
            if nxt < NCHUNKS:
                if g >= 1:
                    out_copy(g - 1).wait()
                in_copy(nxt).start()
        for g in range(NCHUNKS - NBUF, NCHUNKS):
            if g >= 0:
                out_copy(g).wait()

    return k(xf, seg2, flag)


def _tc_body(flag_ref, x_ref, seg_ref, o_ref):
    f = flag_ref[0] != 0
    row = jnp.where(f, seg_ref[0, :], seg_ref[1, :])
    o_ref[...] = x_ref[...] + row[None, :]


def _tc_add(xf, seg2, flag1):
    grid = (TC_ROWS // TC_BLK,)
    return pl.pallas_call(
        _tc_body,
        grid_spec=pltpu.PrefetchScalarGridSpec(
            num_scalar_prefetch=1,
            grid=grid,
            in_specs=[
                pl.BlockSpec((TC_BLK, NUM_HIDDENS),
                             lambda i, flag: (i + TC_OFF_BLKS, 0)),
                pl.BlockSpec((2, NUM_HIDDENS), lambda i, flag: (0, 0)),
            ],
            out_specs=pl.BlockSpec((TC_BLK, NUM_HIDDENS),
                                   lambda i, flag: (i, 0)),
        ),
        out_shape=jax.ShapeDtypeStruct((TC_ROWS, NUM_HIDDENS), jnp.float32),
    )(flag1, xf, seg2)


def kernel(X, seg_emb, first_sentence):
    xf = X.reshape(ROWS, NUM_HIDDENS)
    seg2 = seg_emb.reshape(2, NUM_HIDDENS)
    flag = jnp.full((LANES,), first_sentence, dtype=jnp.int32)
    flag1 = jnp.reshape(jnp.asarray(first_sentence, dtype=jnp.int32), (1,))
    sc_out = _sc_add(xf, seg2, flag)
    tc_out = _tc_add(xf, seg2, flag1)
    out = jnp.concatenate([sc_out, tc_out], axis=0)
    return out.reshape(X.shape)


# speedup vs baseline: 1.5059x; 1.5059x over previous
import functools

import jax
import jax.numpy as jnp
from jax import lax
from jax.experimental import pallas as pl
from jax.experimental.pallas import tpu as pltpu
from jax.experimental.pallas import tpu_sc as plsc

NUM_HIDDENS = 768
LANES = 16
SEG_SLICES = NUM_HIDDENS // LANES
NC, NS = 2, 16
NW = NC * NS
ROWS = 4 * 8192
ROWS_PER_W = ROWS // NW
CHUNK = 32
HALF = CHUNK // 2
NBUF = 4
NCHUNKS = ROWS_PER_W // CHUNK


def _sc_add(xf, seg2, flag):
    mesh = plsc.VectorSubcoreMesh(core_axis_name="c", subcore_axis_name="s")

    @functools.partial(
        pl.kernel,
        mesh=mesh,
        out_type=jax.ShapeDtypeStruct((ROWS, NUM_HIDDENS), jnp.float32),
        scratch_types=[
            pltpu.VMEM((2, NUM_HIDDENS), jnp.float32),
            pltpu.VMEM((LANES,), jnp.int32),
        ] + [pltpu.VMEM((CHUNK, NUM_HIDDENS), jnp.float32)] * NBUF
          + [pltpu.SemaphoreType.DMA] * (2 * NBUF),
    )
    def k(x_hbm, seg_hbm, flag_hbm, out_hbm, seg_v, flag_v, *ring):
        bufs = ring[:NBUF]
        in_sems = ring[NBUF:2 * NBUF]
        out_sems = ring[2 * NBUF:]
        wid = lax.axis_index("s") * NC + lax.axis_index("c")
        pltpu.sync_copy(seg_hbm, seg_v)
        pltpu.sync_copy(flag_hbm, flag_v)
        f = flag_v[...] != 0
        segs = [
            jnp.where(f, seg_v[0, pl.ds(j * LANES, LANES)],
                      seg_v[1, pl.ds(j * LANES, LANES)])
            for j in range(SEG_SLICES)
        ]
        row0 = wid * ROWS_PER_W

        def in_copy(g, b):
            return pltpu.make_async_copy(
                x_hbm.at[pl.ds(row0 + g * CHUNK, CHUNK)], bufs[b], in_sems[b])

        def out_half(g, b, h):
            return pltpu.make_async_copy(
                bufs[b].at[pl.ds(h * HALF, HALF)],
                out_hbm.at[pl.ds(row0 + g * CHUNK + h * HALF, HALF)],
                out_sems[b])

        def compute_half(b, h):
            buf = bufs[b]

            def row_body(r, c):
                for j in range(SEG_SLICES):
                    sl = pl.ds(j * LANES, LANES)
                    plsc.addupdate(buf.at[r, sl], segs[j])
                return c

            lax.fori_loop(h * HALF, (h + 1) * HALF, row_body, 0)

        def process(g, b):
            in_copy(g, b).wait()
            compute_half(b, 0)
            out_half(g, b, 0).start()
            compute_half(b, 1)
            out_half(g, b, 1).start()

        def wait_out(g, b):
            out_half(g, b, 0).wait()
            out_half(g, b, 1).wait()

        for g in range(NBUF - 1):
            in_copy(g, g).start()
        for g in range(NBUF):
            process(g, g)
            if g >= 1:
                wait_out(g - 1, g - 1)
            in_copy(g + NBUF - 1, (g + NBUF - 1) % NBUF).start()

        def group_body(p, c):
            for b in range(NBUF):
                g = p * NBUF + b
                process(g, b)
                wait_out(g - 1, (b - 1) % NBUF)
                in_copy(g + NBUF - 1, (b - 1) % NBUF).start()
            return c

        lax.fori_loop(1, NCHUNKS // NBUF - 1, group_body, 0)

        for g in range(NCHUNKS - NBUF, NCHUNKS):
            b = g % NBUF
            process(g, b)
            wait_out(g - 1, (b - 1) % NBUF)
            if g + NBUF - 1 < NCHUNKS:
                in_copy(g + NBUF - 1, (b - 1) % NBUF).start()
        wait_out(NCHUNKS - 1, (NCHUNKS - 1) % NBUF)

    return k(xf, seg2, flag)


def kernel(X, seg_emb, first_sentence):
    xf = X.reshape(ROWS, NUM_HIDDENS)
    seg2 = seg_emb.reshape(2, NUM_HIDDENS)
    flag = jnp.full((LANES,), first_sentence, dtype=jnp.int32)
    out = _sc_add(xf, seg2, flag)
    return out.reshape(X.shape)
